# baseline (device time: 29741 ns/iter reference)
import math

import jax
import jax.numpy as jnp
from jax import lax
from jax.experimental import pallas as pl
from jax.experimental.pallas import tpu as pltpu

N_DEV = 4


def kernel(q, k, v):
    S, D = q.shape

    def body(q_ref, k_ref, v_ref, out_ref, comm_ref, send_sems, recv_sems):
        my = lax.axis_index("i")
        left = (my + N_DEV - 1) % N_DEV
        right = (my + 1) % N_DEV

        barrier_sem = pltpu.get_barrier_semaphore()
        for nbr in (left, right):
            pl.semaphore_signal(
                barrier_sem, inc=1,
                device_id=(nbr,), device_id_type=pl.DeviceIdType.MESH,
            )
        pl.semaphore_wait(barrier_sem, 2)

        scale = 1.0 / math.sqrt(D)
        q_s = (q_ref[...] * scale).astype(jnp.bfloat16)
        comm_ref[0, :, :D] = k_ref[...].astype(jnp.bfloat16)
        comm_ref[0, :, D:] = v_ref[...].astype(jnp.bfloat16)

        m = jnp.full((S, 1), -jnp.inf, jnp.float32)
        l = jnp.zeros((S, 1), jnp.float32)
        acc = jnp.zeros((S, D), jnp.float32)

        for h in range(N_DEV):
            if h < N_DEV - 1:
                rdma = pltpu.make_async_remote_copy(
                    src_ref=comm_ref.at[h],
                    dst_ref=comm_ref.at[h + 1],
                    send_sem=send_sems.at[h],
                    recv_sem=recv_sems.at[h],
                    device_id=(right,),
                    device_id_type=pl.DeviceIdType.MESH,
                )
                rdma.start()

            kv = comm_ref[h]
            kb = kv[:, :D]
            vb = kv[:, D:]
            s = lax.dot_general(
                q_s, kb, (((1,), (1,)), ((), ())),
                preferred_element_type=jnp.float32,
            )
            m_new = jnp.maximum(m, jnp.max(s, axis=1, keepdims=True))
            p = jnp.exp(s - m_new)
            corr = jnp.exp(m - m_new)
            l = l * corr + jnp.sum(p, axis=1, keepdims=True)
            acc = acc * corr + lax.dot_general(
                p.astype(jnp.bfloat16), vb, (((1,), (0,)), ((), ())),
                preferred_element_type=jnp.float32,
            )
            m = m_new

            if h < N_DEV - 1:
                rdma.wait()

        out_ref[...] = acc / l

    return pl.pallas_call(
        body,
        out_shape=jax.ShapeDtypeStruct((S, D), jnp.float32),
        in_specs=[pl.BlockSpec(memory_space=pltpu.VMEM)] * 3,
        out_specs=pl.BlockSpec(memory_space=pltpu.VMEM),
        scratch_shapes=[
            pltpu.VMEM((N_DEV, S, 2 * D), jnp.bfloat16),
            pltpu.SemaphoreType.DMA((N_DEV - 1,)),
            pltpu.SemaphoreType.DMA((N_DEV - 1,)),
        ],
        compiler_params=pltpu.CompilerParams(collective_id=0),
    )(q, k, v)


# device time: 27783 ns/iter; 1.0705x vs baseline; 1.0705x over previous
import math

import jax
import jax.numpy as jnp
from jax import lax
from jax.experimental import pallas as pl
from jax.experimental.pallas import tpu as pltpu

N_DEV = 4
N_CHUNK = 4


def kernel(q, k, v):
    S, D = q.shape
    R = S // N_CHUNK

    def body(q_ref, k_ref, v_ref, out_ref, comm_ref, send_sems, recv_sems):
        my = lax.axis_index("i")
        left = (my + N_DEV - 1) % N_DEV
        right = (my + 1) % N_DEV

        barrier_sem = pltpu.get_barrier_semaphore()
        for nbr in (left, right):
            pl.semaphore_signal(
                barrier_sem, inc=1,
                device_id=(nbr,), device_id_type=pl.DeviceIdType.MESH,
            )
        pl.semaphore_wait(barrier_sem, 2)

        scale = 1.0 / math.sqrt(D)
        q_s = (q_ref[...] * scale).astype(jnp.bfloat16)
        comm_ref[0, :, :D] = k_ref[...].astype(jnp.bfloat16)
        comm_ref[0, :, D:] = v_ref[...].astype(jnp.bfloat16)

        def chunk_rdma(h, c):
            return pltpu.make_async_remote_copy(
                src_ref=comm_ref.at[h, pl.ds(c * R, R), :],
                dst_ref=comm_ref.at[h + 1, pl.ds(c * R, R), :],
                send_sem=send_sems.at[h, c],
                recv_sem=recv_sems.at[h, c],
                device_id=(right,),
                device_id_type=pl.DeviceIdType.MESH,
            )

        m = jnp.full((S, 1), -jnp.inf, jnp.float32)
        l = jnp.zeros((S, 1), jnp.float32)
        acc = jnp.zeros((S, D), jnp.float32)

        def accumulate(h, m, l, acc):
            kv = comm_ref[h]
            kb = kv[:, :D]
            vb = kv[:, D:]
            s = lax.dot_general(
                q_s, kb, (((1,), (1,)), ((), ())),
                preferred_element_type=jnp.float32,
            )
            m_new = jnp.maximum(m, jnp.max(s, axis=1, keepdims=True))
            p = jnp.exp(s - m_new)
            corr = jnp.exp(m - m_new)
            l = l * corr + jnp.sum(p, axis=1, keepdims=True)
            acc = acc * corr + lax.dot_general(
                p.astype(jnp.bfloat16), vb, (((1,), (0,)), ((), ())),
                preferred_element_type=jnp.float32,
            )
            return m_new, l, acc

        for h in range(N_DEV - 1):
            for c in range(N_CHUNK):
                if h > 0:
                    chunk_rdma(h - 1, c).wait_recv()
                chunk_rdma(h, c).start()
            m, l, acc = accumulate(h, m, l, acc)

        for c in range(N_CHUNK):
            chunk_rdma(N_DEV - 2, c).wait_recv()
        m, l, acc = accumulate(N_DEV - 1, m, l, acc)

        out_ref[...] = acc / l

        for h in range(N_DEV - 1):
            for c in range(N_CHUNK):
                chunk_rdma(h, c).wait_send()

    return pl.pallas_call(
        body,
        out_shape=jax.ShapeDtypeStruct((S, D), jnp.float32),
        in_specs=[pl.BlockSpec(memory_space=pltpu.VMEM)] * 3,
        out_specs=pl.BlockSpec(memory_space=pltpu.VMEM),
        scratch_shapes=[
            pltpu.VMEM((N_DEV, S, 2 * D), jnp.bfloat16),
            pltpu.SemaphoreType.DMA((N_DEV - 1, N_CHUNK)),
            pltpu.SemaphoreType.DMA((N_DEV - 1, N_CHUNK)),
        ],
        compiler_params=pltpu.CompilerParams(collective_id=0),
    )(q, k, v)


# device time: 25578 ns/iter; 1.1628x vs baseline; 1.0862x over previous
import math

import jax
import jax.numpy as jnp
from jax import lax
from jax.experimental import pallas as pl
from jax.experimental.pallas import tpu as pltpu

N_DEV = 4
N_CHUNK = 4


def kernel(q, k, v):
    S, D = q.shape
    R = S // N_CHUNK

    def body(q_ref, k_ref, v_ref, out_ref, comm_ref, send_sems, recv_sems):
        my = lax.axis_index("i")
        left = (my + N_DEV - 1) % N_DEV
        right = (my + 1) % N_DEV

        barrier_sem = pltpu.get_barrier_semaphore()
        for nbr in (left, right):
            pl.semaphore_signal(
                barrier_sem, inc=1,
                device_id=(nbr,), device_id_type=pl.DeviceIdType.MESH,
            )
        pl.semaphore_wait(barrier_sem, 2)

        scale = 1.0 / math.sqrt(D)
        q_s = (q_ref[...] * scale).astype(jnp.bfloat16)
        comm_ref[0, :, :D] = k_ref[...].astype(jnp.bfloat16)
        comm_ref[0, :, D:] = v_ref[...].astype(jnp.bfloat16)

        def chunk_rdma(h, c):
            return pltpu.make_async_remote_copy(
                src_ref=comm_ref.at[h, pl.ds(c * R, R), :],
                dst_ref=comm_ref.at[h + 1, pl.ds(c * R, R), :],
                send_sem=send_sems.at[h, c],
                recv_sem=recv_sems.at[h, c],
                device_id=(right,),
                device_id_type=pl.DeviceIdType.MESH,
            )

        m = jnp.full((S, 1), -jnp.inf, jnp.float32)
        l = jnp.zeros((S, 1), jnp.float32)
        acc = jnp.zeros((S, D), jnp.float32)

        def accumulate_chunk(slot, c, m, l, acc):
            kv = comm_ref[slot, pl.ds(c * R, R), :]
            kb = kv[:, :D]
            vb = kv[:, D:]
            s = lax.dot_general(
                q_s, kb, (((1,), (1,)), ((), ())),
                preferred_element_type=jnp.float32,
            )
            m_new = jnp.maximum(m, jnp.max(s, axis=1, keepdims=True))
            p = jnp.exp(s - m_new)
            corr = jnp.exp(m - m_new)
            l = l * corr + jnp.sum(p, axis=1, keepdims=True)
            acc = acc * corr + lax.dot_general(
                p.astype(jnp.bfloat16), vb, (((1,), (0,)), ((), ())),
                preferred_element_type=jnp.float32,
            )
            return m_new, l, acc

        for c in range(N_CHUNK):
            chunk_rdma(0, c).start()
        for c in range(N_CHUNK):
            m, l, acc = accumulate_chunk(0, c, m, l, acc)

        for slot in range(1, N_DEV):
            for c in range(N_CHUNK):
                chunk_rdma(slot - 1, c).wait_recv()
                if slot < N_DEV - 1:
                    chunk_rdma(slot, c).start()
                m, l, acc = accumulate_chunk(slot, c, m, l, acc)

        out_ref[...] = acc / l

        for h in range(N_DEV - 1):
            for c in range(N_CHUNK):
                chunk_rdma(h, c).wait_send()

    return pl.pallas_call(
        body,
        out_shape=jax.ShapeDtypeStruct((S, D), jnp.float32),
        in_specs=[pl.BlockSpec(memory_space=pltpu.VMEM)] * 3,
        out_specs=pl.BlockSpec(memory_space=pltpu.VMEM),
        scratch_shapes=[
            pltpu.VMEM((N_DEV, S, 2 * D), jnp.bfloat16),
            pltpu.SemaphoreType.DMA((N_DEV - 1, N_CHUNK)),
            pltpu.SemaphoreType.DMA((N_DEV - 1, N_CHUNK)),
        ],
        compiler_params=pltpu.CompilerParams(collective_id=0),
    )(q, k, v)
